# Initial kernel scaffold; baseline (speedup 1.0000x reference)
import jax
import jax.numpy as jnp
from jax.experimental import pallas as pl


def kernel(sequence, vals):
    raise NotImplementedError("write your pallas kernel here")


# speedup vs baseline: 1.3089x; 1.3089x over previous
import functools

import jax
import jax.numpy as jnp
from jax import lax
from jax.experimental import pallas as pl
from jax.experimental.pallas import tpu as pltpu
from jax.experimental.pallas import tpu_sc as plsc

_SEQ_LEN = 16384
_VOCAB = 1000
_PAD = 0

_NC = 2
_NS = 16
_L = 16
_NW = _NC * _NS
_ROWS_PER_W = _SEQ_LEN // _NW
_CH = 64
_CHW = _CH * _VOCAB
_NCHUNK = _ROWS_PER_W // _CH


def _one_hot_body(seq_hbm, vals_hbm, out_hbm, seq_v, vals_v, buf):
    wid = lax.axis_index("s") * _NC + lax.axis_index("c")
    base = wid * _ROWS_PER_W

    pltpu.sync_copy(seq_hbm.at[pl.ds(base, _ROWS_PER_W)], seq_v)
    pltpu.sync_copy(vals_hbm.at[pl.ds(base, _ROWS_PER_W)], vals_v)

    zeros16 = jnp.zeros((_L,), jnp.float32)
    lane = lax.iota(jnp.int32, _L)

    def zbody(i, carry):
        for u in range(8):
            buf[pl.ds((i * 8 + u) * _L, _L)] = zeros16
        return carry

    lax.fori_loop(0, _CHW // (_L * 8), zbody, 0)

    def cbody(c, carry):
        def fill(g, inner):
            r0 = c * _CH + g * _L
            seq16 = seq_v[pl.ds(r0, _L)]
            v16 = vals_v[pl.ds(r0, _L)]
            idx = (g * _L + lane) * _VOCAB + seq16
            plsc.store_scatter(buf, [idx], v16, mask=seq16 != _PAD)
            return inner

        lax.fori_loop(0, _CH // _L, fill, 0)

        pltpu.sync_copy(buf, out_hbm.at[pl.ds((base + c * _CH) * _VOCAB, _CHW)])

        def clear(g, inner):
            r0 = c * _CH + g * _L
            seq16 = seq_v[pl.ds(r0, _L)]
            idx = (g * _L + lane) * _VOCAB + seq16
            plsc.store_scatter(buf, [idx], zeros16)
            return inner

        lax.fori_loop(0, _CH // _L, clear, 0)
        return carry

    lax.fori_loop(0, _NCHUNK, cbody, 0)


@jax.jit
def kernel(sequence, vals):
    mesh = plsc.VectorSubcoreMesh(core_axis_name="c", subcore_axis_name="s")
    flat = pl.kernel(
        _one_hot_body,
        mesh=mesh,
        compiler_params=pltpu.CompilerParams(needs_layout_passes=False),
        out_type=jax.ShapeDtypeStruct((_SEQ_LEN * _VOCAB,), jnp.float32),
        scratch_types=[
            pltpu.VMEM((_ROWS_PER_W,), jnp.int32),
            pltpu.VMEM((_ROWS_PER_W,), jnp.float32),
            pltpu.VMEM((_CHW,), jnp.float32),
        ],
    )(sequence, vals)
    return flat.reshape(_SEQ_LEN, _VOCAB)
